# baseline (speedup 1.0000x reference)
import functools

import jax
import jax.numpy as jnp
from jax import lax
from jax.experimental import pallas as pl
from jax.experimental.pallas import tpu as pltpu
from jax.experimental.pallas import tpu_sc as plsc

N = 10000
D = 128
E = 160000

NC = 2
NS = 16
NW = NC * NS

E_PAD = 163840
ROWS_PER_TILE = E_PAD // (NW * 128)
N_PAD = 10112
ACC_ROWS_PER_TILE = N_PAD // NS
DW = 16


@functools.cache
def _make_agg():
  mesh = plsc.VectorSubcoreMesh(core_axis_name="c", subcore_axis_name="s")

  scratch = [
      pltpu.VMEM((ROWS_PER_TILE, 128), jnp.int32),
      pltpu.VMEM((ROWS_PER_TILE, 128), jnp.int32),
      pltpu.VMEM((128, D), jnp.float32),
      pltpu.VMEM((128, D), jnp.float32),
      pltpu.VMEM_SHARED((N_PAD, D), jnp.float32),
      pltpu.SemaphoreType.DMA,
      pltpu.SemaphoreType.DMA,
  ]

  @functools.partial(
      pl.kernel, mesh=mesh,
      out_type=jax.ShapeDtypeStruct((NC, N_PAD, D), jnp.float32),
      scratch_types=scratch,
  )
  def agg(x_hbm, src_hbm, dst_hbm, zeros_hbm, out_hbm,
          src_v, dst_v, rows0, rows1, acc_sh, sem0, sem1):
    c = lax.axis_index("c")
    s = lax.axis_index("s")
    wid = s * NC + c
    arow = s * ACC_ROWS_PER_TILE

    pltpu.sync_copy(zeros_hbm.at[pl.ds(arow, ACC_ROWS_PER_TILE)],
                    acc_sh.at[pl.ds(arow, ACC_ROWS_PER_TILE)])

    base = wid * ROWS_PER_TILE
    pltpu.sync_copy(src_hbm.at[pl.ds(base, ROWS_PER_TILE)], src_v)
    pltpu.sync_copy(dst_hbm.at[pl.ds(base, ROWS_PER_TILE)], dst_v)

    plsc.subcore_barrier()

    def body(j, carry):
      pltpu.async_copy(x_hbm.at[src_v.at[j]], rows0, sem0).wait()
      pltpu.sync_copy(rows0, acc_sh.at[dst_v.at[j]], add=True)
      return carry

    lax.fori_loop(0, ROWS_PER_TILE, body, 0)

    plsc.subcore_barrier()

    pltpu.sync_copy(acc_sh.at[pl.ds(arow, ACC_ROWS_PER_TILE)],
                    out_hbm.at[c, pl.ds(arow, ACC_ROWS_PER_TILE)])

  return agg


@functools.cache
def _make_deg():
  mesh = plsc.VectorSubcoreMesh(core_axis_name="c", subcore_axis_name="s")

  scratch = [
      pltpu.VMEM((ROWS_PER_TILE, 128), jnp.int32),
      pltpu.VMEM((128, D), jnp.float32),
      pltpu.VMEM_SHARED((N_PAD, D), jnp.float32),
  ]

  @functools.partial(
      pl.kernel, mesh=mesh,
      out_type=jax.ShapeDtypeStruct((NC, N_PAD, D), jnp.float32),
      scratch_types=scratch)
  def deg(dst_hbm, zeros_hbm, ones_hbm, out_hbm, dst_v, ones_v, acc_sh):
    c = lax.axis_index("c")
    s = lax.axis_index("s")
    wid = s * NC + c
    arow = s * ACC_ROWS_PER_TILE

    pltpu.sync_copy(zeros_hbm.at[pl.ds(arow, ACC_ROWS_PER_TILE)],
                    acc_sh.at[pl.ds(arow, ACC_ROWS_PER_TILE)])
    pltpu.sync_copy(ones_hbm, ones_v)

    base = wid * ROWS_PER_TILE
    pltpu.sync_copy(dst_hbm.at[pl.ds(base, ROWS_PER_TILE)], dst_v)

    plsc.subcore_barrier()

    def body(j, carry):
      pltpu.sync_copy(ones_v, acc_sh.at[dst_v.at[j]], add=True)
      return carry

    lax.fori_loop(0, ROWS_PER_TILE, body, 0)

    plsc.subcore_barrier()

    pltpu.sync_copy(acc_sh.at[pl.ds(arow, ACC_ROWS_PER_TILE)],
                    out_hbm.at[c, pl.ds(arow, ACC_ROWS_PER_TILE)])

  return deg


BR = 2528


def _dense_body(x_ref, mhf_ref, dhf_ref, mtt_ref, dtt_ref,
                ws_hf_ref, wn_hf_ref, b_hf_ref,
                ws_tt_ref, wn_tt_ref, b_tt_ref, out_ref):
  x = x_ref[...]

  def rel(m_ref, d_ref, ws_ref, wn_ref, b_ref):
    msum = m_ref[0] + m_ref[1]
    deg = jnp.maximum(d_ref[0] + d_ref[1], 1.0)
    mean = msum / deg
    pre = (jnp.dot(x, ws_ref[...], preferred_element_type=jnp.float32)
           + jnp.dot(mean, wn_ref[...], preferred_element_type=jnp.float32)
           + b_ref[...])
    return jnp.maximum(pre, 0.0)

  out_ref[...] = (rel(mhf_ref, dhf_ref, ws_hf_ref, wn_hf_ref, b_hf_ref)
                  + rel(mtt_ref, dtt_ref, ws_tt_ref, wn_tt_ref, b_tt_ref))


def _dense(x, mhf, dhf, mtt, dtt, ws_hf, wn_hf, b_hf, ws_tt, wn_tt, b_tt):
  grid = (N_PAD // BR,)
  row_blk = pl.BlockSpec((BR, D), lambda i: (i, 0))
  part_blk = pl.BlockSpec((NC, BR, D), lambda i: (0, i, 0))
  deg_blk = pl.BlockSpec((NC, BR, D), lambda i: (0, i, 0))
  w_blk = pl.BlockSpec((D, D), lambda i: (0, 0))
  b_blk = pl.BlockSpec((1, D), lambda i: (0, 0))
  return pl.pallas_call(
      _dense_body,
      grid=grid,
      in_specs=[row_blk, part_blk, deg_blk, part_blk, deg_blk,
                w_blk, w_blk, b_blk, w_blk, w_blk, b_blk],
      out_specs=row_blk,
      out_shape=jax.ShapeDtypeStruct((N_PAD, D), jnp.float32),
  )(x, mhf, dhf, mtt, dtt, ws_hf, wn_hf, b_hf.reshape(1, D),
    ws_tt, wn_tt, b_tt.reshape(1, D))


def _prep_edges(ei):
  pad = E_PAD - E
  src = jnp.concatenate([ei[0], jnp.zeros((pad,), jnp.int32)])
  dst = jnp.concatenate([ei[1], jnp.full((pad,), N, jnp.int32)])
  return src.reshape(E_PAD // 128, 128), dst.reshape(E_PAD // 128, 128)


def kernel(h, edge_index_hf, edge_index_tt,
           Ws_0_hf, Wn_0_hf, b_0_hf, Ws_0_tt, Wn_0_tt, b_0_tt,
           Ws_1_hf, Wn_1_hf, b_1_hf, Ws_1_tt, Wn_1_tt, b_1_tt,
           Ws_2_hf, Wn_2_hf, b_2_hf, Ws_2_tt, Wn_2_tt, b_2_tt):
  src_hf, dst_hf = _prep_edges(edge_index_hf)
  src_tt, dst_tt = _prep_edges(edge_index_tt)

  zeros = jnp.zeros((N_PAD, D), jnp.float32)
  ones_tile = jnp.ones((128, D), jnp.float32)

  agg = _make_agg()
  deg_k = _make_deg()

  x = jnp.concatenate([h, jnp.zeros((N_PAD - N, D), jnp.float32)])

  dhf = deg_k(dst_hf, zeros, ones_tile)
  dtt = deg_k(dst_tt, zeros, ones_tile)

  for (ws_hf, wn_hf, b_hf, ws_tt, wn_tt, b_tt) in [
      (Ws_0_hf, Wn_0_hf, b_0_hf, Ws_0_tt, Wn_0_tt, b_0_tt),
      (Ws_1_hf, Wn_1_hf, b_1_hf, Ws_1_tt, Wn_1_tt, b_1_tt),
      (Ws_2_hf, Wn_2_hf, b_2_hf, Ws_2_tt, Wn_2_tt, b_2_tt),
  ]:
    mhf = agg(x, src_hf, dst_hf, zeros)
    mtt = agg(x, src_tt, dst_tt, zeros)
    x = _dense(x, mhf, dhf, mtt, dtt, ws_hf, wn_hf, b_hf, ws_tt, wn_tt, b_tt)

  return x[:N]


# speedup vs baseline: 1.1332x; 1.1332x over previous
import functools

import jax
import jax.numpy as jnp
from jax import lax
from jax.experimental import pallas as pl
from jax.experimental.pallas import tpu as pltpu
from jax.experimental.pallas import tpu_sc as plsc

N = 10000
D = 128
E = 160000

NC = 2
NS = 16
NW = NC * NS

E_PAD = 163840
ROWS_PER_TILE = E_PAD // (NW * 128)
N_PAD = 10112
ACC_ROWS_PER_TILE = N_PAD // NS
DW = 16


@functools.cache
def _make_agg():
  mesh = plsc.VectorSubcoreMesh(core_axis_name="c", subcore_axis_name="s")

  scratch = [
      pltpu.VMEM((ROWS_PER_TILE, 128), jnp.int32),
      pltpu.VMEM((ROWS_PER_TILE, 128), jnp.int32),
      pltpu.VMEM((128, D), jnp.float32),
      pltpu.VMEM((128, D), jnp.float32),
      pltpu.VMEM_SHARED((N_PAD, D), jnp.float32),
      pltpu.SemaphoreType.DMA,
      pltpu.SemaphoreType.DMA,
  ]

  @functools.partial(
      pl.kernel, mesh=mesh,
      out_type=jax.ShapeDtypeStruct((NC, N_PAD, D), jnp.float32),
      scratch_types=scratch,
  )
  def agg(x_hbm, src_hbm, dst_hbm, zeros_hbm, out_hbm,
          src_v, dst_v, rows0, rows1, acc_sh, sem0, sem1):
    c = lax.axis_index("c")
    s = lax.axis_index("s")
    wid = s * NC + c
    arow = s * ACC_ROWS_PER_TILE

    pltpu.sync_copy(zeros_hbm.at[pl.ds(arow, ACC_ROWS_PER_TILE)],
                    acc_sh.at[pl.ds(arow, ACC_ROWS_PER_TILE)])

    base = wid * ROWS_PER_TILE
    pltpu.sync_copy(src_hbm.at[pl.ds(base, ROWS_PER_TILE)], src_v)
    pltpu.sync_copy(dst_hbm.at[pl.ds(base, ROWS_PER_TILE)], dst_v)

    plsc.subcore_barrier()

    pltpu.async_copy(x_hbm.at[src_v.at[0]], rows0, sem0)
    pltpu.async_copy(x_hbm.at[src_v.at[1]], rows1, sem1)

    def halfstep(j, rows, sem):
      pltpu.make_async_copy(x_hbm.at[src_v.at[j]], rows, sem).wait()
      pltpu.sync_copy(rows, acc_sh.at[dst_v.at[j]], add=True)

      @pl.when(j + 2 < ROWS_PER_TILE)
      def _():
        pltpu.async_copy(x_hbm.at[src_v.at[j + 2]], rows, sem)

    def body(i, carry):
      halfstep(2 * i, rows0, sem0)
      halfstep(2 * i + 1, rows1, sem1)
      return carry

    lax.fori_loop(0, ROWS_PER_TILE // 2, body, 0)

    plsc.subcore_barrier()

    pltpu.sync_copy(acc_sh.at[pl.ds(arow, ACC_ROWS_PER_TILE)],
                    out_hbm.at[c, pl.ds(arow, ACC_ROWS_PER_TILE)])

  return agg


@functools.cache
def _make_deg():
  mesh = plsc.VectorSubcoreMesh(core_axis_name="c", subcore_axis_name="s")

  scratch = [
      pltpu.VMEM((ROWS_PER_TILE, 128), jnp.int32),
      pltpu.VMEM((128, D), jnp.float32),
      pltpu.VMEM_SHARED((N_PAD, D), jnp.float32),
  ]

  @functools.partial(
      pl.kernel, mesh=mesh,
      out_type=jax.ShapeDtypeStruct((NC, N_PAD, D), jnp.float32),
      scratch_types=scratch)
  def deg(dst_hbm, zeros_hbm, ones_hbm, out_hbm, dst_v, ones_v, acc_sh):
    c = lax.axis_index("c")
    s = lax.axis_index("s")
    wid = s * NC + c
    arow = s * ACC_ROWS_PER_TILE

    pltpu.sync_copy(zeros_hbm.at[pl.ds(arow, ACC_ROWS_PER_TILE)],
                    acc_sh.at[pl.ds(arow, ACC_ROWS_PER_TILE)])
    pltpu.sync_copy(ones_hbm, ones_v)

    base = wid * ROWS_PER_TILE
    pltpu.sync_copy(dst_hbm.at[pl.ds(base, ROWS_PER_TILE)], dst_v)

    plsc.subcore_barrier()

    def body(j, carry):
      pltpu.sync_copy(ones_v, acc_sh.at[dst_v.at[j]], add=True)
      return carry

    lax.fori_loop(0, ROWS_PER_TILE, body, 0)

    plsc.subcore_barrier()

    pltpu.sync_copy(acc_sh.at[pl.ds(arow, ACC_ROWS_PER_TILE)],
                    out_hbm.at[c, pl.ds(arow, ACC_ROWS_PER_TILE)])

  return deg


BR = 2528


def _dense_body(x_ref, mhf_ref, dhf_ref, mtt_ref, dtt_ref,
                ws_hf_ref, wn_hf_ref, b_hf_ref,
                ws_tt_ref, wn_tt_ref, b_tt_ref, out_ref):
  x = x_ref[...]

  def rel(m_ref, d_ref, ws_ref, wn_ref, b_ref):
    msum = m_ref[0] + m_ref[1]
    deg = jnp.maximum(d_ref[0] + d_ref[1], 1.0)
    mean = msum / deg
    pre = (jnp.dot(x, ws_ref[...], preferred_element_type=jnp.float32)
           + jnp.dot(mean, wn_ref[...], preferred_element_type=jnp.float32)
           + b_ref[...])
    return jnp.maximum(pre, 0.0)

  out_ref[...] = (rel(mhf_ref, dhf_ref, ws_hf_ref, wn_hf_ref, b_hf_ref)
                  + rel(mtt_ref, dtt_ref, ws_tt_ref, wn_tt_ref, b_tt_ref))


def _dense(x, mhf, dhf, mtt, dtt, ws_hf, wn_hf, b_hf, ws_tt, wn_tt, b_tt):
  grid = (N_PAD // BR,)
  row_blk = pl.BlockSpec((BR, D), lambda i: (i, 0))
  part_blk = pl.BlockSpec((NC, BR, D), lambda i: (0, i, 0))
  deg_blk = pl.BlockSpec((NC, BR, D), lambda i: (0, i, 0))
  w_blk = pl.BlockSpec((D, D), lambda i: (0, 0))
  b_blk = pl.BlockSpec((1, D), lambda i: (0, 0))
  return pl.pallas_call(
      _dense_body,
      grid=grid,
      in_specs=[row_blk, part_blk, deg_blk, part_blk, deg_blk,
                w_blk, w_blk, b_blk, w_blk, w_blk, b_blk],
      out_specs=row_blk,
      out_shape=jax.ShapeDtypeStruct((N_PAD, D), jnp.float32),
  )(x, mhf, dhf, mtt, dtt, ws_hf, wn_hf, b_hf.reshape(1, D),
    ws_tt, wn_tt, b_tt.reshape(1, D))


def _prep_edges(ei):
  pad = E_PAD - E
  src = jnp.concatenate([ei[0], jnp.zeros((pad,), jnp.int32)])
  dst = jnp.concatenate([ei[1], jnp.full((pad,), N, jnp.int32)])
  return src.reshape(E_PAD // 128, 128), dst.reshape(E_PAD // 128, 128)


def kernel(h, edge_index_hf, edge_index_tt,
           Ws_0_hf, Wn_0_hf, b_0_hf, Ws_0_tt, Wn_0_tt, b_0_tt,
           Ws_1_hf, Wn_1_hf, b_1_hf, Ws_1_tt, Wn_1_tt, b_1_tt,
           Ws_2_hf, Wn_2_hf, b_2_hf, Ws_2_tt, Wn_2_tt, b_2_tt):
  src_hf, dst_hf = _prep_edges(edge_index_hf)
  src_tt, dst_tt = _prep_edges(edge_index_tt)

  zeros = jnp.zeros((N_PAD, D), jnp.float32)
  ones_tile = jnp.ones((128, D), jnp.float32)

  agg = _make_agg()
  deg_k = _make_deg()

  x = jnp.concatenate([h, jnp.zeros((N_PAD - N, D), jnp.float32)])

  dhf = deg_k(dst_hf, zeros, ones_tile)
  dtt = deg_k(dst_tt, zeros, ones_tile)

  for (ws_hf, wn_hf, b_hf, ws_tt, wn_tt, b_tt) in [
      (Ws_0_hf, Wn_0_hf, b_0_hf, Ws_0_tt, Wn_0_tt, b_0_tt),
      (Ws_1_hf, Wn_1_hf, b_1_hf, Ws_1_tt, Wn_1_tt, b_1_tt),
      (Ws_2_hf, Wn_2_hf, b_2_hf, Ws_2_tt, Wn_2_tt, b_2_tt),
  ]:
    mhf = agg(x, src_hf, dst_hf, zeros)
    mtt = agg(x, src_tt, dst_tt, zeros)
    x = _dense(x, mhf, dhf, mtt, dtt, ws_hf, wn_hf, b_hf, ws_tt, wn_tt, b_tt)

  return x[:N]


# speedup vs baseline: 1.2650x; 1.1163x over previous
import functools

import jax
import jax.numpy as jnp
from jax import lax
from jax.experimental import pallas as pl
from jax.experimental.pallas import tpu as pltpu
from jax.experimental.pallas import tpu_sc as plsc

N = 10000
D = 128
E = 160000

NC = 2
NS = 16

E_PAD = 163840
ROWS_PER_TILE = E_PAD // (NS * 128)
CHUNK = ROWS_PER_TILE // 2
N_PAD = 10112
ACC_ROWS_PER_TILE = N_PAD // NS


@functools.cache
def _make_agg():
  mesh = plsc.VectorSubcoreMesh(core_axis_name="c", subcore_axis_name="s")

  NBUF = 2
  scratch = (
      [pltpu.VMEM((CHUNK, 128), jnp.int32)] * 2
      + [pltpu.VMEM((128, D), jnp.float32)] * NBUF
      + [pltpu.VMEM_SHARED((N_PAD, D), jnp.float32)]
      + [pltpu.SemaphoreType.DMA] * NBUF
  )

  @functools.partial(
      pl.kernel, mesh=mesh,
      out_type=jax.ShapeDtypeStruct((NC, N_PAD, D), jnp.float32),
      scratch_types=scratch,
  )
  def agg(x_hbm, src_hbm, dst_hbm, zeros_hbm, out_hbm,
          src_v, dst_v, *bufs_and_sems):
    rows = bufs_and_sems[:NBUF]
    acc_sh = bufs_and_sems[NBUF]
    sems = bufs_and_sems[NBUF + 1:]
    c = lax.axis_index("c")
    s = lax.axis_index("s")
    arow = s * ACC_ROWS_PER_TILE

    pltpu.sync_copy(zeros_hbm.at[pl.ds(arow, ACC_ROWS_PER_TILE)],
                    acc_sh.at[pl.ds(arow, ACC_ROWS_PER_TILE)])

    plsc.subcore_barrier()

    for chunk in range(ROWS_PER_TILE // CHUNK):
      base = s * ROWS_PER_TILE + chunk * CHUNK
      pltpu.sync_copy(src_hbm.at[c, pl.ds(base, CHUNK)], src_v)
      pltpu.sync_copy(dst_hbm.at[c, pl.ds(base, CHUNK)], dst_v)

      for b in range(NBUF):
        pltpu.async_copy(x_hbm.at[src_v.at[b]], rows[b], sems[b])

      def substep(j, buf, sem):
        pltpu.make_async_copy(x_hbm.at[src_v.at[j]], buf, sem).wait()
        pltpu.sync_copy(buf, acc_sh.at[dst_v.at[j]], add=True)

        @pl.when(j + NBUF < CHUNK)
        def _():
          pltpu.async_copy(x_hbm.at[src_v.at[j + NBUF]], buf, sem)

      def body(i, carry):
        for b in range(NBUF):
          substep(NBUF * i + b, rows[b], sems[b])
        return carry

      lax.fori_loop(0, CHUNK // NBUF, body, 0)

    plsc.subcore_barrier()

    pltpu.sync_copy(acc_sh.at[pl.ds(arow, ACC_ROWS_PER_TILE)],
                    out_hbm.at[c, pl.ds(arow, ACC_ROWS_PER_TILE)])

  return agg


@functools.cache
def _make_deg():
  mesh = plsc.VectorSubcoreMesh(core_axis_name="c", subcore_axis_name="s")

  scratch = [
      pltpu.VMEM((ROWS_PER_TILE, 128), jnp.int32),
      pltpu.VMEM((128, D), jnp.float32),
      pltpu.VMEM_SHARED((N_PAD, D), jnp.float32),
  ]

  @functools.partial(
      pl.kernel, mesh=mesh,
      out_type=jax.ShapeDtypeStruct((NC, N_PAD, D), jnp.float32),
      scratch_types=scratch)
  def deg(dst_hbm, zeros_hbm, ones_hbm, out_hbm, dst_v, ones_v, acc_sh):
    c = lax.axis_index("c")
    s = lax.axis_index("s")
    arow = s * ACC_ROWS_PER_TILE

    pltpu.sync_copy(zeros_hbm.at[pl.ds(arow, ACC_ROWS_PER_TILE)],
                    acc_sh.at[pl.ds(arow, ACC_ROWS_PER_TILE)])
    pltpu.sync_copy(ones_hbm, ones_v)

    base = s * ROWS_PER_TILE
    pltpu.sync_copy(dst_hbm.at[c, pl.ds(base, ROWS_PER_TILE)], dst_v)

    plsc.subcore_barrier()

    def body(j, carry):
      pltpu.sync_copy(ones_v, acc_sh.at[dst_v.at[j]], add=True)
      return carry

    lax.fori_loop(0, ROWS_PER_TILE, body, 0)

    plsc.subcore_barrier()

    pltpu.sync_copy(acc_sh.at[pl.ds(arow, ACC_ROWS_PER_TILE)],
                    out_hbm.at[c, pl.ds(arow, ACC_ROWS_PER_TILE)])

  return deg


BR = 2528


def _dense_body(x_ref, m_ref, d_ref,
                ws_hf_ref, wn_hf_ref, b_hf_ref,
                ws_tt_ref, wn_tt_ref, b_tt_ref, out_ref):
  x = x_ref[...]

  def rel(r, ws_ref, wn_ref, b_ref):
    mean = m_ref[r] / jnp.maximum(d_ref[r], 1.0)
    pre = (jnp.dot(x, ws_ref[...], preferred_element_type=jnp.float32)
           + jnp.dot(mean, wn_ref[...], preferred_element_type=jnp.float32)
           + b_ref[...])
    return jnp.maximum(pre, 0.0)

  out_ref[...] = (rel(0, ws_hf_ref, wn_hf_ref, b_hf_ref)
                  + rel(1, ws_tt_ref, wn_tt_ref, b_tt_ref))


def _dense(x, m, d, ws_hf, wn_hf, b_hf, ws_tt, wn_tt, b_tt):
  grid = (N_PAD // BR,)
  row_blk = pl.BlockSpec((BR, D), lambda i: (i, 0))
  part_blk = pl.BlockSpec((NC, BR, D), lambda i: (0, i, 0))
  w_blk = pl.BlockSpec((D, D), lambda i: (0, 0))
  b_blk = pl.BlockSpec((1, D), lambda i: (0, 0))
  return pl.pallas_call(
      _dense_body,
      grid=grid,
      in_specs=[row_blk, part_blk, part_blk,
                w_blk, w_blk, b_blk, w_blk, w_blk, b_blk],
      out_specs=row_blk,
      out_shape=jax.ShapeDtypeStruct((N_PAD, D), jnp.float32),
  )(x, m, d, ws_hf, wn_hf, b_hf.reshape(1, D),
    ws_tt, wn_tt, b_tt.reshape(1, D))


def _prep_edges(ei):
  pad = E_PAD - E
  src = jnp.concatenate([ei[0], jnp.zeros((pad,), jnp.int32)])
  dst = jnp.concatenate([ei[1], jnp.full((pad,), N, jnp.int32)])
  return src.reshape(E_PAD // 128, 128), dst.reshape(E_PAD // 128, 128)


def kernel(h, edge_index_hf, edge_index_tt,
           Ws_0_hf, Wn_0_hf, b_0_hf, Ws_0_tt, Wn_0_tt, b_0_tt,
           Ws_1_hf, Wn_1_hf, b_1_hf, Ws_1_tt, Wn_1_tt, b_1_tt,
           Ws_2_hf, Wn_2_hf, b_2_hf, Ws_2_tt, Wn_2_tt, b_2_tt):
  src_hf, dst_hf = _prep_edges(edge_index_hf)
  src_tt, dst_tt = _prep_edges(edge_index_tt)
  src = jnp.stack([src_hf, src_tt])
  dst = jnp.stack([dst_hf, dst_tt])

  zeros = jnp.zeros((N_PAD, D), jnp.float32)
  ones_tile = jnp.ones((128, D), jnp.float32)

  agg = _make_agg()
  deg_k = _make_deg()

  x = jnp.concatenate([h, jnp.zeros((N_PAD - N, D), jnp.float32)])

  d = deg_k(dst, zeros, ones_tile)

  for (ws_hf, wn_hf, b_hf, ws_tt, wn_tt, b_tt) in [
      (Ws_0_hf, Wn_0_hf, b_0_hf, Ws_0_tt, Wn_0_tt, b_0_tt),
      (Ws_1_hf, Wn_1_hf, b_1_hf, Ws_1_tt, Wn_1_tt, b_1_tt),
      (Ws_2_hf, Wn_2_hf, b_2_hf, Ws_2_tt, Wn_2_tt, b_2_tt),
  ]:
    m = agg(x, src, dst, zeros)
    x = _dense(x, m, d, ws_hf, wn_hf, b_hf, ws_tt, wn_tt, b_tt)

  return x[:N]
